# baseline (speedup 1.0000x reference)
import functools

import jax
import jax.numpy as jnp
from jax import lax
from jax.experimental import pallas as pl
from jax.experimental.pallas import tpu as pltpu
from jax.experimental.pallas import tpu_sc as plsc

_BLK = 512
_RING = 64


@functools.lru_cache(maxsize=None)
def _make_gather(V, D, B):
    info = plsc.get_sparse_core_info()
    NC, NS = info.num_cores, info.num_subcores
    NW = NC * NS
    V_pad = -(-V // 128) * 128
    s_last = V_pad - _BLK
    n_blk = -(-V_pad // (_BLK * NW))
    span = n_blk * _BLK
    mesh = plsc.VectorSubcoreMesh(core_axis_name="c", subcore_axis_name="s")

    @functools.partial(
        pl.kernel,
        mesh=mesh,
        out_type=jax.ShapeDtypeStruct((B, D), jnp.float32),
        compiler_params=pltpu.CompilerParams(needs_layout_passes=False),
        scratch_types=[
            pltpu.VMEM((B,), jnp.int32),
            pltpu.VMEM((B,), jnp.int32),
            pltpu.VMEM((B + 16,), jnp.int32),
            pltpu.VMEM((2, 8, _BLK // 128, D // 8, 128), jnp.float32),
            pltpu.VMEM((_RING, D), jnp.float32),
            pltpu.SemaphoreType.DMA,
            pltpu.SemaphoreType.DMA,
            pltpu.SemaphoreType.DMA,
        ],
    )
    def gather_kernel(
        tab_hbm, idx_hbm, out_hbm,
        lab_v, mp_v, pk_v, blk_v, row_v, sem0, sem1, osem,
    ):
        wid = lax.axis_index("s") * NC + lax.axis_index("c")
        lo = wid * span
        hi = lo + span
        lanes = lax.iota(jnp.int32, 16)

        pltpu.sync_copy(idx_hbm, lab_v)

        def scan_body(i, count):
            vec = lab_v[pl.ds(i * 16, 16)]
            pos = lanes + i * 16
            m = (vec >= lo) & (vec < hi)
            plsc.store_compressed(lab_v.at[pl.ds(count, 16)], vec, mask=m)
            plsc.store_compressed(mp_v.at[pl.ds(count, 16)], pos, mask=m)
            n = plsc.all_reduce_population_count(m)
            return count + (n if n.ndim == 0 else n[0])

        k = lax.fori_loop(0, B // 16, scan_body, jnp.int32(0))
        n_chunks = (k + 15) // 16

        def blk_start(b):
            s = jnp.minimum(lo + b * _BLK, s_last)
            return pl.multiple_of(s, 128)

        n_tc4 = _BLK // 128

        def start_block(b, u, sem):
            s0 = blk_start(b)
            for tr in range(D // 8):
                for tc in range(n_tc4):
                    pltpu.async_copy(
                        tab_hbm.at[
                            pl.ds(tr * 8, 8),
                            pl.ds(pl.multiple_of(s0 + tc * 128, 128), 128),
                        ],
                        blk_v.at[u, tr, tc],
                        sem,
                    )

        def wait_block(b, u, sem):
            s0 = blk_start(b)
            for tr in range(D // 8):
                for tc in range(n_tc4):
                    pltpu.make_async_copy(
                        tab_hbm.at[
                            pl.ds(tr * 8, 8),
                            pl.ds(pl.multiple_of(s0 + tc * 128, 128), 128),
                        ],
                        blk_v.at[u, tr, tc],
                        sem,
                    ).wait()

        start_block(0, 0, sem0)
        start_block(1, 1, sem1)

        def drain(n_out):
            def d1(_, c):
                pltpu.make_async_copy(
                    row_v.at[pl.ds(0, 1)], out_hbm.at[pl.ds(0, 1)], osem
                ).wait()
                return c

            lax.fori_loop(0, n_out, d1, 0)

        def outer(g, n_out):
            for u, sem in ((0, sem0), (1, sem1)):
                b = g * 2 + u
                nom = lo + b * _BLK
                s = blk_start(b)
                wait_block(b, u, sem)

                def pha(c, ke):
                    vec = lab_v[pl.ds(c * 16, 16)]
                    pv = mp_v[pl.ds(c * 16, 16)]
                    m = (vec >= nom) & (vec < nom + _BLK) & (lanes + c * 16 < k)
                    packed = (vec - nom) | (pv << 9)
                    plsc.store_compressed(pk_v.at[pl.ds(ke, 16)], packed, mask=m)
                    nv = plsc.all_reduce_population_count(m)
                    return ke + (nv if nv.ndim == 0 else nv[0])

                ke = lax.fori_loop(0, n_chunks, pha, jnp.int32(0))
                dsh = nom - s

                def phb(e, st):
                    x = pk_v[pl.ds(e, 16)][0]
                    col = (x & (_BLK - 1)) + dsh
                    pos = x >> 9
                    tcq = lanes * 0 + (col >> 7)
                    oc = lanes * 0 + (col & 127)
                    need = st >= _RING

                    @pl.when(need)
                    def _():
                        drain(_RING)

                    st = jnp.where(need, 0, st)
                    for q in range(D // 16):
                        cvec = lanes + q * 16
                        r = plsc.load_gather(
                            blk_v.at[u],
                            [cvec >> 3, tcq, cvec & 7, oc],
                        )
                        row_v[st, pl.ds(q * 16, 16)] = r
                    pltpu.async_copy(
                        row_v.at[pl.ds(st, 1)],
                        out_hbm.at[pl.ds(pos, 1)],
                        osem,
                    )
                    return st + 1

                n_out = lax.fori_loop(0, ke, phb, n_out)
                nxt = b + 2

                @pl.when(nxt < n_blk)
                def _():
                    start_block(nxt, u, sem)

            return n_out

        n_out = lax.fori_loop(0, n_blk // 2, outer, jnp.int32(0))
        drain(n_out)

    return gather_kernel, NW


def kernel(labels, train, embedding_table):
    del train
    B = labels.shape[0]
    V, D = embedding_table.shape
    fn, NW = _make_gather(V, D, B)
    return fn(embedding_table.T, labels.astype(jnp.int32))


# speedup vs baseline: 1.0542x; 1.0542x over previous
import functools

import jax
import jax.numpy as jnp
from jax import lax
from jax.experimental import pallas as pl
from jax.experimental.pallas import tpu as pltpu
from jax.experimental.pallas import tpu_sc as plsc

_BLK = 256
_NBUF = 4
_RING = 64


@functools.lru_cache(maxsize=None)
def _make_gather(V, D, B):
    info = plsc.get_sparse_core_info()
    NC, NS = info.num_cores, info.num_subcores
    NW = NC * NS
    V_pad = -(-V // 128) * 128
    s_last = V_pad - _BLK
    n_blk = -(-V_pad // (_BLK * NW))
    n_blk = -(-n_blk // _NBUF) * _NBUF
    span = n_blk * _BLK
    mesh = plsc.VectorSubcoreMesh(core_axis_name="c", subcore_axis_name="s")

    @functools.partial(
        pl.kernel,
        mesh=mesh,
        out_type=jax.ShapeDtypeStruct((B, D), jnp.float32),
        compiler_params=pltpu.CompilerParams(needs_layout_passes=False),
        scratch_types=[
            pltpu.VMEM((B,), jnp.int32),
            pltpu.VMEM((B,), jnp.int32),
            pltpu.VMEM((B + 16,), jnp.int32),
            pltpu.VMEM((_NBUF, 8, _BLK // 128, D // 8, 128), jnp.float32),
            pltpu.VMEM((_RING, D), jnp.float32),
        ] + [pltpu.SemaphoreType.DMA] * (_NBUF + 1),
    )
    def gather_kernel(
        tab_hbm, idx_hbm, out_hbm,
        lab_v, mp_v, pk_v, blk_v, row_v, *sems,
    ):
        wid = lax.axis_index("s") * NC + lax.axis_index("c")
        lo = wid * span
        hi = lo + span
        lanes = lax.iota(jnp.int32, 16)

        pltpu.sync_copy(idx_hbm, lab_v)

        def scan_body(i, count):
            vec = lab_v[pl.ds(i * 16, 16)]
            pos = lanes + i * 16
            m = (vec >= lo) & (vec < hi)
            plsc.store_compressed(lab_v.at[pl.ds(count, 16)], vec, mask=m)
            plsc.store_compressed(mp_v.at[pl.ds(count, 16)], pos, mask=m)
            n = plsc.all_reduce_population_count(m)
            return count + (n if n.ndim == 0 else n[0])

        k = lax.fori_loop(0, B // 16, scan_body, jnp.int32(0))
        n_chunks = (k + 15) // 16

        def blk_start(b):
            s = jnp.minimum(lo + b * _BLK, s_last)
            return pl.multiple_of(s, 128)

        n_tc4 = _BLK // 128

        def start_block(b, u, sem):
            s0 = blk_start(b)
            for tr in range(D // 8):
                for tc in range(n_tc4):
                    pltpu.async_copy(
                        tab_hbm.at[
                            pl.ds(tr * 8, 8),
                            pl.ds(pl.multiple_of(s0 + tc * 128, 128), 128),
                        ],
                        blk_v.at[u, tr, tc],
                        sem,
                    )

        def wait_block(b, u, sem):
            s0 = blk_start(b)
            for tr in range(D // 8):
                for tc in range(n_tc4):
                    pltpu.make_async_copy(
                        tab_hbm.at[
                            pl.ds(tr * 8, 8),
                            pl.ds(pl.multiple_of(s0 + tc * 128, 128), 128),
                        ],
                        blk_v.at[u, tr, tc],
                        sem,
                    ).wait()

        osem = sems[_NBUF]

        for u in range(_NBUF):
            start_block(u, u, sems[u])

        def drain(n_out):
            def d1(_, c):
                pltpu.make_async_copy(
                    row_v.at[pl.ds(0, 1)], out_hbm.at[pl.ds(0, 1)], osem
                ).wait()
                return c

            lax.fori_loop(0, n_out, d1, 0)

        def outer(g, n_out):
            for u in range(_NBUF):
                sem = sems[u]
                b = g * _NBUF + u
                nom = lo + b * _BLK
                s = blk_start(b)
                wait_block(b, u, sem)

                def pha(c, ke):
                    vec = lab_v[pl.ds(c * 16, 16)]
                    pv = mp_v[pl.ds(c * 16, 16)]
                    m = (vec >= nom) & (vec < nom + _BLK) & (lanes + c * 16 < k)
                    packed = (vec - nom) | (pv << 9)
                    plsc.store_compressed(pk_v.at[pl.ds(ke, 16)], packed, mask=m)
                    nv = plsc.all_reduce_population_count(m)
                    return ke + (nv if nv.ndim == 0 else nv[0])

                ke = lax.fori_loop(0, n_chunks, pha, jnp.int32(0))
                dsh = nom - s

                def phb(e, st):
                    x = pk_v[pl.ds(e, 16)][0]
                    col = (x & (_BLK - 1)) + dsh
                    pos = x >> 9
                    tcq = lanes * 0 + (col >> 7)
                    oc = lanes * 0 + (col & 127)
                    need = st >= _RING

                    @pl.when(need)
                    def _():
                        drain(_RING)

                    st = jnp.where(need, 0, st)
                    for q in range(D // 16):
                        cvec = lanes + q * 16
                        r = plsc.load_gather(
                            blk_v.at[u],
                            [cvec >> 3, tcq, cvec & 7, oc],
                        )
                        row_v[st, pl.ds(q * 16, 16)] = r
                    pltpu.async_copy(
                        row_v.at[pl.ds(st, 1)],
                        out_hbm.at[pl.ds(pos, 1)],
                        osem,
                    )
                    return st + 1

                n_out = lax.fori_loop(0, ke, phb, n_out)
                nxt = b + _NBUF

                @pl.when(nxt < n_blk)
                def _():
                    start_block(nxt, u, sem)

            return n_out

        n_out = lax.fori_loop(0, n_blk // _NBUF, outer, jnp.int32(0))
        drain(n_out)

    return gather_kernel, NW


def kernel(labels, train, embedding_table):
    del train
    B = labels.shape[0]
    V, D = embedding_table.shape
    fn, NW = _make_gather(V, D, B)
    return fn(embedding_table.T, labels.astype(jnp.int32))


# speedup vs baseline: 1.0839x; 1.0282x over previous
import functools

import jax
import jax.numpy as jnp
from jax import lax
from jax.experimental import pallas as pl
from jax.experimental.pallas import tpu as pltpu
from jax.experimental.pallas import tpu_sc as plsc

_BLK = 256
_NBUF = 4
_RING = 64


@functools.lru_cache(maxsize=None)
def _make_gather(V, D, B):
    info = plsc.get_sparse_core_info()
    NC, NS = info.num_cores, info.num_subcores
    NW = NC * NS
    V_pad = -(-V // 128) * 128
    s_last = V_pad - _BLK
    n_blk = -(-V_pad // (_BLK * NW))
    n_blk = -(-n_blk // _NBUF) * _NBUF
    span = n_blk * _BLK
    mesh = plsc.VectorSubcoreMesh(core_axis_name="c", subcore_axis_name="s")

    @functools.partial(
        pl.kernel,
        mesh=mesh,
        out_type=jax.ShapeDtypeStruct((B, D), jnp.float32),
        compiler_params=pltpu.CompilerParams(needs_layout_passes=False),
        scratch_types=[
            pltpu.VMEM((B,), jnp.int32),
            pltpu.VMEM((B,), jnp.int32),
            pltpu.VMEM((B + 16,), jnp.int32),
            pltpu.VMEM((_NBUF, D // 8, 8, _BLK), jnp.float32),
            pltpu.VMEM((_RING, D), jnp.float32),
        ] + [pltpu.SemaphoreType.DMA] * (_NBUF + 1),
    )
    def gather_kernel(
        tab_hbm, idx_hbm, out_hbm,
        lab_v, mp_v, pk_v, blk_v, row_v, *sems,
    ):
        wid = lax.axis_index("s") * NC + lax.axis_index("c")
        lo = wid * span
        hi = lo + span
        lanes = lax.iota(jnp.int32, 16)

        pltpu.sync_copy(idx_hbm, lab_v)

        def scan_body(i, count):
            vec = lab_v[pl.ds(i * 16, 16)]
            pos = lanes + i * 16
            m = (vec >= lo) & (vec < hi)
            plsc.store_compressed(lab_v.at[pl.ds(count, 16)], vec, mask=m)
            plsc.store_compressed(mp_v.at[pl.ds(count, 16)], pos, mask=m)
            n = plsc.all_reduce_population_count(m)
            return count + (n if n.ndim == 0 else n[0])

        k = lax.fori_loop(0, B // 16, scan_body, jnp.int32(0))
        n_chunks = (k + 15) // 16

        def blk_start(b):
            s = jnp.minimum(lo + b * _BLK, s_last)
            return pl.multiple_of(s, 128)

        n_tc4 = _BLK // 128

        def start_block(b, u, sem):
            s0 = blk_start(b)
            for tr in range(D // 8):
                pltpu.async_copy(
                    tab_hbm.at[
                        pl.ds(tr * 8, 8),
                        pl.ds(pl.multiple_of(s0, 128), _BLK),
                    ],
                    blk_v.at[u, tr],
                    sem,
                )

        def wait_block(b, u, sem):
            s0 = blk_start(b)
            for tr in range(D // 8):
                pltpu.make_async_copy(
                    tab_hbm.at[
                        pl.ds(tr * 8, 8),
                        pl.ds(pl.multiple_of(s0, 128), _BLK),
                    ],
                    blk_v.at[u, tr],
                    sem,
                ).wait()

        osem = sems[_NBUF]

        for u in range(_NBUF):
            start_block(u, u, sems[u])

        def drain(n_out):
            def d1(_, c):
                pltpu.make_async_copy(
                    row_v.at[pl.ds(0, 1)], out_hbm.at[pl.ds(0, 1)], osem
                ).wait()
                return c

            lax.fori_loop(0, n_out, d1, 0)

        def outer(g, n_out):
            for u in range(_NBUF):
                sem = sems[u]
                b = g * _NBUF + u
                nom = lo + b * _BLK
                s = blk_start(b)
                wait_block(b, u, sem)

                def pha(c, ke):
                    vec = lab_v[pl.ds(c * 16, 16)]
                    pv = mp_v[pl.ds(c * 16, 16)]
                    m = (vec >= nom) & (vec < nom + _BLK) & (lanes + c * 16 < k)
                    packed = (vec - nom) | (pv << 9)
                    plsc.store_compressed(pk_v.at[pl.ds(ke, 16)], packed, mask=m)
                    nv = plsc.all_reduce_population_count(m)
                    return ke + (nv if nv.ndim == 0 else nv[0])

                ke = lax.fori_loop(0, n_chunks, pha, jnp.int32(0))
                dsh = nom - s

                def phb(e, st):
                    x = pk_v[pl.ds(e, 16)][0]
                    col = (x & (_BLK - 1)) + dsh
                    pos = x >> 9
                    jsp = lanes * 0 + col
                    need = st >= _RING

                    @pl.when(need)
                    def _():
                        drain(_RING)

                    st = jnp.where(need, 0, st)
                    for q in range(D // 16):
                        cvec = lanes + q * 16
                        r = plsc.load_gather(
                            blk_v.at[u],
                            [cvec >> 3, cvec & 7, jsp],
                        )
                        row_v[st, pl.ds(q * 16, 16)] = r
                    pltpu.async_copy(
                        row_v.at[pl.ds(st, 1)],
                        out_hbm.at[pl.ds(pos, 1)],
                        osem,
                    )
                    return st + 1

                n_out = lax.fori_loop(0, ke, phb, n_out)
                nxt = b + _NBUF

                @pl.when(nxt < n_blk)
                def _():
                    start_block(nxt, u, sem)

            return n_out

        n_out = lax.fori_loop(0, n_blk // _NBUF, outer, jnp.int32(0))
        drain(n_out)

    return gather_kernel, NW


def kernel(labels, train, embedding_table):
    del train
    B = labels.shape[0]
    V, D = embedding_table.shape
    fn, NW = _make_gather(V, D, B)
    return fn(embedding_table.T, labels.astype(jnp.int32))


# speedup vs baseline: 1.0841x; 1.0002x over previous
import functools

import jax
import jax.numpy as jnp
from jax import lax
from jax.experimental import pallas as pl
from jax.experimental.pallas import tpu as pltpu
from jax.experimental.pallas import tpu_sc as plsc

_BLK = 256
_NBUF = 4
_RING = 64


@functools.lru_cache(maxsize=None)
def _make_gather(V, D, B):
    info = plsc.get_sparse_core_info()
    NC, NS = info.num_cores, info.num_subcores
    NW = NC * NS
    V_pad = -(-V // 128) * 128
    s_last = V_pad - _BLK
    n_blk = -(-V_pad // (_BLK * NW))
    n_blk = -(-n_blk // _NBUF) * _NBUF
    span = n_blk * _BLK
    mesh = plsc.VectorSubcoreMesh(core_axis_name="c", subcore_axis_name="s")

    @functools.partial(
        pl.kernel,
        mesh=mesh,
        out_type=jax.ShapeDtypeStruct((B, D), jnp.float32),
        compiler_params=pltpu.CompilerParams(needs_layout_passes=False),
        scratch_types=[
            pltpu.VMEM((B,), jnp.int32),
            pltpu.VMEM((B,), jnp.int32),
            pltpu.VMEM((B + 16,), jnp.int32),
            pltpu.VMEM((_NBUF, D // 8, 8, _BLK), jnp.float32),
            pltpu.VMEM((_RING, D), jnp.float32),
        ] + [pltpu.SemaphoreType.DMA] * (_NBUF + 1),
    )
    def gather_kernel(
        tab_hbm, idx_hbm, out_hbm,
        lab_v, mp_v, pk_v, blk_v, row_v, *sems,
    ):
        wid = lax.axis_index("s") * NC + lax.axis_index("c")
        lo = wid * span
        hi = lo + span
        lanes = lax.iota(jnp.int32, 16)

        pltpu.sync_copy(idx_hbm, lab_v)

        def scan_body(i, count):
            vec = lab_v[pl.ds(i * 16, 16)]
            pos = lanes + i * 16
            m = (vec >= lo) & (vec < hi)
            plsc.store_compressed(lab_v.at[pl.ds(count, 16)], vec, mask=m)
            plsc.store_compressed(mp_v.at[pl.ds(count, 16)], pos, mask=m)
            n = plsc.all_reduce_population_count(m)
            return count + (n if n.ndim == 0 else n[0])

        k = lax.fori_loop(0, B // 16, scan_body, jnp.int32(0))
        n_chunks = (k + 15) // 16

        def blk_start(b):
            s = jnp.minimum(lo + b * _BLK, s_last)
            return pl.multiple_of(s, 128)

        def start_block(b, u, sem):
            s0 = blk_start(b)
            for tr in range(D // 8):
                pltpu.async_copy(
                    tab_hbm.at[
                        pl.ds(tr * 8, 8),
                        pl.ds(pl.multiple_of(s0, 128), _BLK),
                    ],
                    blk_v.at[u, tr],
                    sem,
                )

        def wait_block(b, u, sem):
            s0 = blk_start(b)
            for tr in range(D // 8):
                pltpu.make_async_copy(
                    tab_hbm.at[
                        pl.ds(tr * 8, 8),
                        pl.ds(pl.multiple_of(s0, 128), _BLK),
                    ],
                    blk_v.at[u, tr],
                    sem,
                ).wait()

        osem = sems[_NBUF]

        for u in range(_NBUF):
            start_block(u, u, sems[u])

        def drain(n_out):
            def d1(_, c):
                pltpu.make_async_copy(
                    row_v.at[pl.ds(0, 1)], out_hbm.at[pl.ds(0, 1)], osem
                ).wait()
                return c

            lax.fori_loop(0, n_out, d1, 0)

        def outer(g, n_out):
            for u in range(_NBUF):
                sem = sems[u]
                b = g * _NBUF + u
                nom = lo + b * _BLK
                s = blk_start(b)
                wait_block(b, u, sem)

                def pha(c, ke):
                    vec = lab_v[pl.ds(c * 16, 16)]
                    pv = mp_v[pl.ds(c * 16, 16)]
                    m = (vec >= nom) & (vec < nom + _BLK) & (lanes + c * 16 < k)
                    packed = (vec - nom) | (pv << 9)
                    plsc.store_compressed(pk_v.at[pl.ds(ke, 16)], packed, mask=m)
                    nv = plsc.all_reduce_population_count(m)
                    return ke + (nv if nv.ndim == 0 else nv[0])

                ke = lax.fori_loop(0, n_chunks, pha, jnp.int32(0))
                dsh = nom - s

                def phb(e, st):
                    x = pk_v[pl.ds(e, 16)][0]
                    col = (x & (_BLK - 1)) + dsh
                    pos = x >> 9
                    jsp = lanes * 0 + col
                    need = st >= _RING

                    @pl.when(need)
                    def _():
                        drain(_RING)

                    st = jnp.where(need, 0, st)
                    for q in range(D // 16):
                        cvec = lanes + q * 16
                        r = plsc.load_gather(
                            blk_v.at[u],
                            [cvec >> 3, cvec & 7, jsp],
                        )
                        row_v[st, pl.ds(q * 16, 16)] = r
                    pltpu.async_copy(
                        row_v.at[pl.ds(st, 1)],
                        out_hbm.at[pl.ds(pos, 1)],
                        osem,
                    )
                    return st + 1

                n_out = lax.fori_loop(0, ke, phb, n_out)
                nxt = b + _NBUF

                @pl.when(nxt < n_blk)
                def _():
                    start_block(nxt, u, sem)

            return n_out

        n_out = lax.fori_loop(0, n_blk // _NBUF, outer, jnp.int32(0))
        drain(n_out)

    return gather_kernel, NW


def kernel(labels, train, embedding_table):
    del train
    B = labels.shape[0]
    V, D = embedding_table.shape
    fn, NW = _make_gather(V, D, B)
    return fn(embedding_table.T, labels.astype(jnp.int32))
